# baseline (speedup 1.0000x reference)
import functools

import jax
import jax.numpy as jnp
from jax import lax
from jax.experimental import pallas as pl
from jax.experimental.pallas import tpu as pltpu
from jax.experimental.pallas import tpu_sc as plsc

_E = 16
_K = 64
_R = 768
_A = 16
_ARITY = 3
_TILE = 1024
_N = 32768

_NC = 2
_NS = 16
_NW = _NC * _NS
_PER_W = _N // _NW
_CHUNK = 64
_NCHUNK = _PER_W // _CHUNK


def _centroid_body(atoms_ref, cw_ref, cl_ref, out_ref):
    iota_a = jax.lax.broadcasted_iota(jnp.int32, (_K, _A), 1)
    ab = jnp.sign(atoms_ref[0])
    acc = jnp.zeros((_K, _R), jnp.float32)
    for b in range(_ARITY):
        lg = cl_ref[0, :, b, :]
        z = lg - jnp.max(lg, axis=-1, keepdims=True)
        ez = jnp.exp(z)
        soft = ez / jnp.sum(ez, axis=-1, keepdims=True)
        m = jnp.max(soft, axis=-1, keepdims=True)
        idx = jnp.min(jnp.where(soft == m, iota_a, _A), axis=-1,
                      keepdims=True)
        onehot = (iota_a == idx).astype(jnp.float32)
        sel = jax.lax.dot_general(
            onehot, ab, (((1,), (0,)), ((), ())),
            preferred_element_type=jnp.float32)
        acc = acc + sel * cw_ref[0, :, b:b + 1]
    cw = jnp.sign(acc)
    out_ref[0] = jnp.sum(cw, axis=0, keepdims=True) * (1.0 / _K)


def _router_body(x_ref, w_ref, e0_ref, e1_ref, g0_ref, g1_ref, aux_ref):
    step = pl.program_id(0)
    xt = x_ref[...]
    logits = jax.lax.dot_general(
        xt, w_ref[...], (((1,), (1,)), ((), ())),
        preferred_element_type=jnp.float32)
    lt = jnp.transpose(logits)

    z = lt - jnp.max(lt, axis=0, keepdims=True)
    ez = jnp.exp(z)
    probs = ez / jnp.sum(ez, axis=0, keepdims=True)

    iota_e = jax.lax.broadcasted_iota(jnp.int32, (_E, _TILE), 0)
    m0 = jnp.max(probs, axis=0, keepdims=True)
    i0 = jnp.min(jnp.where(probs == m0, iota_e, _E), axis=0, keepdims=True)
    masked = jnp.where(iota_e == i0, -1.0, probs)
    m1 = jnp.max(masked, axis=0, keepdims=True)
    i1 = jnp.min(jnp.where(masked == m1, iota_e, _E), axis=0, keepdims=True)
    inv = 1.0 / (m0 + m1)
    e0_ref[...] = i0[0]
    e1_ref[...] = i1[0]
    g0_ref[...] = (m0 * inv)[0]
    g1_ref[...] = (m1 * inv)[0]

    col0 = jnp.sum(probs, axis=1, keepdims=True)
    col1 = jnp.sum((probs > 0).astype(jnp.float32), axis=1, keepdims=True)
    aux_val = jnp.concatenate([col0, col1], axis=1)

    @pl.when(step == 0)
    def _():
        aux_ref[...] = jnp.zeros_like(aux_ref)

    aux_ref[...] += aux_val


def _combine_body(e0_hbm, e1_hbm, g0_hbm, g1_hbm, cent_hbm, out_hbm,
                  cent_v, outa_v, outb_v, e0_v, e1_v, g0_v, g1_v, sem):
    wid = lax.axis_index("s") * _NC + lax.axis_index("c")
    tok0 = wid * _PER_W
    pltpu.sync_copy(cent_hbm, cent_v)
    pltpu.sync_copy(e0_hbm.at[pl.ds(tok0, _PER_W)],
                    e0_v.at[pl.ds(0, _PER_W)])
    pltpu.sync_copy(e1_hbm.at[pl.ds(tok0, _PER_W)],
                    e1_v.at[pl.ds(0, _PER_W)])
    pltpu.sync_copy(g0_hbm.at[pl.ds(tok0, _PER_W)],
                    g0_v.at[pl.ds(0, _PER_W)])
    pltpu.sync_copy(g1_hbm.at[pl.ds(tok0, _PER_W)],
                    g1_v.at[pl.ds(0, _PER_W)])

    def process(buf, base):
        @plsc.parallel_loop(0, _CHUNK, 1, unroll=4)
        def _(t):
            tok = base + t
            e0 = e0_v[pl.ds(tok, 16)][0]
            e1 = e1_v[pl.ds(tok, 16)][0]
            g0 = g0_v[pl.ds(tok, 16)][0]
            g1 = g1_v[pl.ds(tok, 16)][0]
            for j in range(_R // 16):
                c0 = cent_v[e0, pl.ds(j * 16, 16)]
                c1 = cent_v[e1, pl.ds(j * 16, 16)]
                buf[t, pl.ds(j * 16, 16)] = c0 * g0 + c1 * g1

    def outer(ch2, carry):
        base = ch2 * (2 * _CHUNK)

        @pl.when(ch2 > 0)
        def _():
            pltpu.make_async_copy(
                outa_v, out_hbm.at[pl.ds(tok0, _CHUNK)], sem).wait()

        process(outa_v, base)
        pltpu.async_copy(
            outa_v, out_hbm.at[pl.ds(tok0 + base, _CHUNK)], sem)

        @pl.when(ch2 > 0)
        def _():
            pltpu.make_async_copy(
                outb_v, out_hbm.at[pl.ds(tok0, _CHUNK)], sem).wait()

        process(outb_v, base + _CHUNK)
        pltpu.async_copy(
            outb_v, out_hbm.at[pl.ds(tok0 + base + _CHUNK, _CHUNK)], sem)
        return carry

    lax.fori_loop(0, _NCHUNK // 2, outer, 0)
    pltpu.make_async_copy(outa_v, out_hbm.at[pl.ds(tok0, _CHUNK)],
                          sem).wait()
    pltpu.make_async_copy(outb_v, out_hbm.at[pl.ds(tok0, _CHUNK)],
                          sem).wait()


@functools.lru_cache(maxsize=1)
def _make_combine():
    return pl.kernel(
        _combine_body,
        out_type=jax.ShapeDtypeStruct((_N, _R), jnp.float32),
        mesh=plsc.VectorSubcoreMesh(core_axis_name="c",
                                    subcore_axis_name="s"),
        scratch_types=[
            pltpu.VMEM((_E, _R), jnp.float32),
            pltpu.VMEM((_CHUNK, _R), jnp.float32),
            pltpu.VMEM((_CHUNK, _R), jnp.float32),
            pltpu.VMEM((_PER_W + 16,), jnp.int32),
            pltpu.VMEM((_PER_W + 16,), jnp.int32),
            pltpu.VMEM((_PER_W + 16,), jnp.float32),
            pltpu.VMEM((_PER_W + 16,), jnp.float32),
            pltpu.SemaphoreType.DMA,
        ],
    )


@jax.jit
def kernel(x_latent, W_router, atoms, combo_weights, combo_logits):
    B, S, R = x_latent.shape
    N = B * S
    x2 = x_latent.reshape(N, R)

    centroids = pl.pallas_call(
        _centroid_body,
        grid=(_E,),
        in_specs=[
            pl.BlockSpec((1, _A, _R), lambda e: (e, 0, 0)),
            pl.BlockSpec((1, _K, _ARITY), lambda e: (e, 0, 0)),
            pl.BlockSpec((1, _K, _ARITY, _A), lambda e: (e, 0, 0, 0)),
        ],
        out_specs=pl.BlockSpec((1, 1, _R), lambda e: (e, 0, 0)),
        out_shape=jax.ShapeDtypeStruct((_E, 1, _R), jnp.float32),
    )(atoms, combo_weights, combo_logits)
    centroids = centroids.reshape(_E, _R)

    grid = N // _TILE
    e0a, e1a, g0a, g1a, aux = pl.pallas_call(
        _router_body,
        grid=(grid,),
        in_specs=[
            pl.BlockSpec((_TILE, R), lambda i: (i, 0)),
            pl.BlockSpec((_E, R), lambda i: (0, 0)),
        ],
        out_specs=[
            pl.BlockSpec((_TILE,), lambda i: (i,)),
            pl.BlockSpec((_TILE,), lambda i: (i,)),
            pl.BlockSpec((_TILE,), lambda i: (i,)),
            pl.BlockSpec((_TILE,), lambda i: (i,)),
            pl.BlockSpec((_E, 2), lambda i: (0, 0)),
        ],
        out_shape=[
            jax.ShapeDtypeStruct((N,), jnp.int32),
            jax.ShapeDtypeStruct((N,), jnp.int32),
            jax.ShapeDtypeStruct((N,), jnp.float32),
            jax.ShapeDtypeStruct((N,), jnp.float32),
            jax.ShapeDtypeStruct((_E, 2), jnp.float32),
        ],
        compiler_params=pltpu.CompilerParams(
            dimension_semantics=("arbitrary",)),
    )(x2, W_router)

    combined = _make_combine()(e0a, e1a, g0a, g1a, centroids)

    inv_n = 1.0 / N
    aux_loss = _E * jnp.sum((aux[:, 0] * inv_n) * (aux[:, 1] * inv_n))
    return combined.reshape(B, S, R), aux_loss


# speedup vs baseline: 2.4117x; 2.4117x over previous
import functools

import jax
import jax.numpy as jnp
from jax.experimental import pallas as pl
from jax.experimental.pallas import tpu as pltpu

_E = 16
_K = 64
_R = 768
_A = 16
_ARITY = 3
_TILE = 2048


def _centroid_body(atoms_ref, cw_ref, cl_ref, out_ref):
    iota_a = jax.lax.broadcasted_iota(jnp.int32, (_K, _A), 1)
    ab = jnp.sign(atoms_ref[0])
    acc = jnp.zeros((_K, _R), jnp.float32)
    for b in range(_ARITY):
        lg = cl_ref[0, :, b, :]
        z = lg - jnp.max(lg, axis=-1, keepdims=True)
        ez = jnp.exp(z)
        soft = ez / jnp.sum(ez, axis=-1, keepdims=True)
        m = jnp.max(soft, axis=-1, keepdims=True)
        idx = jnp.min(jnp.where(soft == m, iota_a, _A), axis=-1,
                      keepdims=True)
        onehot = (iota_a == idx).astype(jnp.float32)
        sel = jax.lax.dot_general(
            onehot, ab, (((1,), (0,)), ((), ())),
            preferred_element_type=jnp.float32)
        acc = acc + sel * cw_ref[0, :, b:b + 1]
    cw = jnp.sign(acc)
    out_ref[0] = jnp.sum(cw, axis=0, keepdims=True) * (1.0 / _K)


def _main_body(x_ref, w_ref, cent_ref, out_ref, aux_ref):
    step = pl.program_id(0)
    xt = x_ref[...]
    logits = jax.lax.dot_general(
        xt, w_ref[...], (((1,), (1,)), ((), ())),
        preferred_element_type=jnp.float32)
    lt = jnp.transpose(logits)

    z = lt - jnp.max(lt, axis=0, keepdims=True)
    ez = jnp.exp(z)
    probs = ez / jnp.sum(ez, axis=0, keepdims=True)

    iota_e = jax.lax.broadcasted_iota(jnp.int32, (_E, _TILE), 0)
    m0 = jnp.max(probs, axis=0, keepdims=True)
    i0 = jnp.min(jnp.where(probs == m0, iota_e, _E), axis=0, keepdims=True)
    masked = jnp.where(iota_e == i0, -1.0, probs)
    m1 = jnp.max(masked, axis=0, keepdims=True)
    i1 = jnp.min(jnp.where(masked == m1, iota_e, _E), axis=0, keepdims=True)
    inv = 1.0 / (m0 + m1)
    eg = jnp.where(iota_e == i0, m0 * inv, 0.0) + jnp.where(
        iota_e == i1, m1 * inv, 0.0)
    out_ref[...] = jax.lax.dot_general(
        eg.astype(jnp.bfloat16), cent_ref[...], (((0,), (0,)), ((), ())),
        preferred_element_type=jnp.float32)

    col0 = jnp.sum(probs, axis=1, keepdims=True)
    col1 = jnp.sum((probs > 0).astype(jnp.float32), axis=1, keepdims=True)
    aux_val = jnp.concatenate([col0, col1], axis=1)

    @pl.when(step == 0)
    def _():
        aux_ref[...] = jnp.zeros_like(aux_ref)

    aux_ref[...] += aux_val


@jax.jit
def kernel(x_latent, W_router, atoms, combo_weights, combo_logits):
    B, S, R = x_latent.shape
    N = B * S
    x2 = x_latent.reshape(N, R)

    centroids = pl.pallas_call(
        _centroid_body,
        grid=(_E,),
        in_specs=[
            pl.BlockSpec((1, _A, _R), lambda e: (e, 0, 0)),
            pl.BlockSpec((1, _K, _ARITY), lambda e: (e, 0, 0)),
            pl.BlockSpec((1, _K, _ARITY, _A), lambda e: (e, 0, 0, 0)),
        ],
        out_specs=pl.BlockSpec((1, 1, _R), lambda e: (e, 0, 0)),
        out_shape=jax.ShapeDtypeStruct((_E, 1, _R), jnp.float32),
    )(atoms, combo_weights, combo_logits)
    cent_bf = centroids.reshape(_E, _R).astype(jnp.bfloat16)

    grid = N // _TILE
    combined, aux = pl.pallas_call(
        _main_body,
        grid=(grid,),
        in_specs=[
            pl.BlockSpec((_TILE, R), lambda i: (i, 0)),
            pl.BlockSpec((_E, R), lambda i: (0, 0)),
            pl.BlockSpec((_E, R), lambda i: (0, 0)),
        ],
        out_specs=[
            pl.BlockSpec((_TILE, R), lambda i: (i, 0)),
            pl.BlockSpec((_E, 2), lambda i: (0, 0)),
        ],
        out_shape=[
            jax.ShapeDtypeStruct((N, R), jnp.float32),
            jax.ShapeDtypeStruct((_E, 2), jnp.float32),
        ],
        compiler_params=pltpu.CompilerParams(
            dimension_semantics=("arbitrary",)),
    )(x2, W_router, cent_bf)

    inv_n = 1.0 / N
    aux_loss = _E * jnp.sum((aux[:, 0] * inv_n) * (aux[:, 1] * inv_n))
    return combined.reshape(B, S, R), aux_loss
